# baseline (speedup 1.0000x reference)
import functools

import jax
import jax.numpy as jnp
from jax import lax
from jax.experimental import pallas as pl
from jax.experimental.pallas import tpu as pltpu, tpu_sc as plsc

NV = 50000
NF = 100000
NE = 150000

NC = 2
NS = 16
NW = NC * NS
LANES = 16
CHUNK = 128

FC = 32
FT = FC * CHUNK
FP = NW * FT
EC = 40
ET = EC * CHUNK
EP = NW * ET
NSUP = 4
ES = ET // NSUP

_ATAN = (0.999998017, -0.333060167, 0.196054925, -0.122270662,
         0.0585597433, -0.0138876227)
PI = 3.14159265358979
PIO2 = PI / 2

_MESH = plsc.VectorSubcoreMesh(core_axis_name="c", subcore_axis_name="s")


def _rsqrt(x):
    i = lax.bitcast_convert_type(x, jnp.int32)
    i = jnp.int32(0x5F3759DF) - (i >> 1)
    y = lax.bitcast_convert_type(i, jnp.float32)
    for _ in range(3):
        y = y * (1.5 - 0.5 * x * y * y)
    return y


def _atan2_sq(s, d):
    p, q = jnp.abs(d), jnp.abs(s)
    mx, mn = jnp.maximum(p, q), jnp.minimum(p, q)
    z = mn / jnp.maximum(mx, 1e-30)
    z2 = z * z
    t = jnp.float32(_ATAN[5])
    for c in _ATAN[4::-1]:
        t = t * z2 + c
    t = t * z
    t = jnp.where(q > p, PIO2 - t, t)
    t = jnp.where(d < 0.0, PI - t, t)
    return t * t


def _wid():
    return lax.axis_index("s") * NC + lax.axis_index("c")


@functools.partial(
    pl.kernel,
    out_type=(jax.ShapeDtypeStruct((FP,), jnp.float32),) * 3,
    mesh=_MESH,
    scratch_types=[pltpu.VMEM_SHARED((NV,), jnp.float32)] * 3
    + [pltpu.VMEM((FT,), jnp.int32)] * 3
    + [pltpu.VMEM((FT,), jnp.float32)] * 9
    + [pltpu.VMEM((FT,), jnp.float32)] * 3
    + [pltpu.SemaphoreType.DMA],
)
def _face_k(px, py, pz, f0, f1, f2, ocx, ocy, ocz,
            spx, spy, spz, i0, i1, i2, *rest):
    bufs = rest[0:9]
    obx, oby, obz = rest[9:12]
    sem = rest[12]
    w = _wid()
    fbase = w * FT

    @pl.when(lax.axis_index("s") == 0)
    def _stage():
        pltpu.sync_copy(px, spx)
        pltpu.sync_copy(py, spy)
        pltpu.sync_copy(pz, spz)

    pltpu.sync_copy(f0.at[pl.ds(fbase, FT)], i0)
    pltpu.sync_copy(f1.at[pl.ds(fbase, FT)], i1)
    pltpu.sync_copy(f2.at[pl.ds(fbase, FT)], i2)
    plsc.subcore_barrier()

    srcs = [spx.at[i0], spy.at[i0], spz.at[i0],
            spx.at[i1], spy.at[i1], spz.at[i1],
            spx.at[i2], spy.at[i2], spz.at[i2]]
    for s, d in zip(srcs, bufs):
        pltpu.async_copy(s, d, sem)
    for s, d in zip(srcs, bufs):
        pltpu.make_async_copy(s, d, sem).wait()

    b0x, b0y, b0z, b1x, b1y, b1z, b2x, b2y, b2z = bufs

    def row(r, carry):
        for sb in range(8):
            sl = pl.ds(r * CHUNK + sb * LANES, LANES)
            ax, ay, az = b0x[sl], b0y[sl], b0z[sl]
            ux, uy, uz = b1x[sl] - ax, b1y[sl] - ay, b1z[sl] - az
            wx, wy, wz = b2x[sl] - ax, b2y[sl] - ay, b2z[sl] - az
            obx[sl] = uy * wz - uz * wy
            oby[sl] = uz * wx - ux * wz
            obz[sl] = ux * wy - uy * wx
        return carry

    lax.fori_loop(0, FC, row, 0, unroll=False)

    dst = pl.ds(fbase, FT)
    pltpu.sync_copy(obx, ocx.at[dst])
    pltpu.sync_copy(oby, ocy.at[dst])
    pltpu.sync_copy(obz, ocz.at[dst])


@functools.partial(
    pl.kernel,
    out_type=jax.ShapeDtypeStruct((NW * LANES,), jnp.float32),
    mesh=_MESH,
    scratch_types=[pltpu.VMEM_SHARED((NV,), jnp.float32)] * 3
    + [pltpu.VMEM_SHARED((FP,), jnp.float32)] * 4
    + [pltpu.VMEM((ET,), jnp.int32)] * 4
    + [pltpu.VMEM((ES,), jnp.float32)] * 28
    + [pltpu.VMEM((LANES,), jnp.float32)]
    + [pltpu.SemaphoreType.DMA] * 2,
)
def _edge_k(px, py, pz, cx, cy, cz, af, jc0, jc1, je0, je1, part_out,
            spx, spy, spz, scx, scy, scz, saf,
            ic0, ic1, ie0, ie1, *rest):
    bufs = [rest[14 * b:14 * b + 14] for b in range(2)]
    accb = rest[28]
    sems = rest[29:31]
    w = _wid()
    ebase = w * ET

    @pl.when(lax.axis_index("s") == 0)
    def _stage():
        pltpu.sync_copy(px, spx)
        pltpu.sync_copy(py, spy)
        pltpu.sync_copy(pz, spz)
        pltpu.sync_copy(cx, scx)
        pltpu.sync_copy(cy, scy)
        pltpu.sync_copy(cz, scz)
        pltpu.sync_copy(af, saf)

    pltpu.sync_copy(jc0.at[pl.ds(ebase, ET)], ic0)
    pltpu.sync_copy(jc1.at[pl.ds(ebase, ET)], ic1)
    pltpu.sync_copy(je0.at[pl.ds(ebase, ET)], ie0)
    pltpu.sync_copy(je1.at[pl.ds(ebase, ET)], ie1)
    plsc.subcore_barrier()

    def srcs(h):
        sl = pl.ds(h * ES, ES)
        ix0, ix1, iv0, iv1 = ic0.at[sl], ic1.at[sl], ie0.at[sl], ie1.at[sl]
        return [scx.at[ix0], scy.at[ix0], scz.at[ix0], saf.at[ix0],
                scx.at[ix1], scy.at[ix1], scz.at[ix1], saf.at[ix1],
                spx.at[iv0], spy.at[iv0], spz.at[iv0],
                spx.at[iv1], spy.at[iv1], spz.at[iv1]]

    def issue(h, b):
        for s, d in zip(srcs(h), bufs[b]):
            pltpu.async_copy(s, d, sems[b])

    def drain(h, b):
        for s, d in zip(srcs(h), bufs[b]):
            pltpu.make_async_copy(s, d, sems[b]).wait()

    def compute(b, acc):
        (g0x, g0y, g0z, g0a, g1x, g1y, g1z, g1a,
         h0x, h0y, h0z, h1x, h1y, h1z) = bufs[b]

        def blk(r, acc):
            sl = pl.ds(r * LANES, LANES)
            c0x, c0y, c0z, a0 = g0x[sl], g0y[sl], g0z[sl], g0a[sl]
            c1x, c1y, c1z, a1 = g1x[sl], g1y[sl], g1z[sl], g1a[sl]
            ex = h1x[sl] - h0x[sl]
            ey = h1y[sl] - h0y[sl]
            ez = h1z[sl] - h0z[sl]
            ll = ex * ex + ey * ey + ez * ez
            dot = c0x * c1x + c0y * c1y + c0z * c1z
            gx = c0y * c1z - c0z * c1y
            gy = c0z * c1x - c0x * c1z
            gz = c0x * c1y - c0y * c1x
            sn = ex * gx + ey * gy + ez * gz
            s = sn * _rsqrt(ll)
            th2 = _atan2_sq(s, dot)
            return acc + th2 * ll / (4.0 * (a0 + a1))

        return lax.fori_loop(0, ES // LANES, blk, acc, unroll=4)

    issue(0, 0)

    def body(t, acc):
        h0 = 2 * t
        issue(h0 + 1, 1)
        drain(h0, 0)
        acc = compute(0, acc)
        pl.when(t + 1 < NSUP // 2)(lambda: issue(h0 + 2, 0))
        drain(h0 + 1, 1)
        acc = compute(1, acc)
        return acc

    acc = lax.fori_loop(0, NSUP // 2, body, jnp.zeros((LANES,), jnp.float32),
                        unroll=False)
    accb[...] = acc
    pltpu.sync_copy(accb, part_out.at[pl.ds(w * LANES, LANES)])


def kernel(pred_pos, faces, f_connectivity, f_connectivity_edges, f_area,
           bending_coeff):
    px, py, pz = pred_pos[:, 0], pred_pos[:, 1], pred_pos[:, 2]
    ft = jnp.pad(faces.T.astype(jnp.int32), ((0, 0), (0, FP - NF)))
    af = jnp.pad(f_area[:, 0], (0, FP - NF))
    fct = jnp.pad(f_connectivity.T.astype(jnp.int32), ((0, 0), (0, EP - NE)))
    fet = jnp.pad(f_connectivity_edges.T.astype(jnp.int32),
                  ((0, 0), (0, EP - NE)))

    cx, cy, cz = _face_k(px, py, pz, ft[0], ft[1], ft[2])
    parts = _edge_k(px, py, pz, cx, cy, cz, af,
                    fct[0], fct[1], fet[0], fet[1])
    return jnp.sum(parts) * (bending_coeff[0] * 0.5)


# speedup vs baseline: 2.0682x; 2.0682x over previous
import functools

import jax
import jax.numpy as jnp
from jax import lax
from jax.experimental import pallas as pl
from jax.experimental.pallas import tpu as pltpu, tpu_sc as plsc

NV = 50000
NF = 100000
NE = 150000

NC = 2
NS = 16
NW = NC * NS
LANES = 16
CHUNK = 128

FC = 25
FT = FC * CHUNK
FP = NW * FT
EC = 37
ET = EC * CHUNK
EP = NW * ET

_ATAN = (0.999998017, -0.333060167, 0.196054925, -0.122270662,
         0.0585597433, -0.0138876227)
PI = 3.14159265358979
PIO2 = PI / 2

_MESH = plsc.VectorSubcoreMesh(core_axis_name="c", subcore_axis_name="s")


def _rsqrt(x):
    i = lax.bitcast_convert_type(x, jnp.int32)
    i = jnp.int32(0x5F3759DF) - (i >> 1)
    y = lax.bitcast_convert_type(i, jnp.float32)
    for _ in range(3):
        y = y * (1.5 - 0.5 * x * y * y)
    return y


def _atan2_sq(s, d):
    p, q = jnp.abs(d), jnp.abs(s)
    mx, mn = jnp.maximum(p, q), jnp.minimum(p, q)
    z = mn / jnp.maximum(mx, 1e-30)
    z2 = z * z
    t = jnp.float32(_ATAN[5])
    for c in _ATAN[4::-1]:
        t = t * z2 + c
    t = t * z
    t = jnp.where(q > p, PIO2 - t, t)
    t = jnp.where(d < 0.0, PI - t, t)
    return t * t


def _wid():
    return lax.axis_index("s") * NC + lax.axis_index("c")


_F32B = pltpu.VMEM((CHUNK,), jnp.float32)


@functools.partial(
    pl.kernel,
    out_type=(jax.ShapeDtypeStruct((FP,), jnp.float32),) * 3,
    mesh=_MESH,
    scratch_types=[pltpu.VMEM_SHARED((NV,), jnp.float32)] * 3
    + [pltpu.VMEM((FT,), jnp.int32)] * 3
    + [_F32B] * 18
    + [pltpu.VMEM((FT,), jnp.float32)] * 3
    + [pltpu.SemaphoreType.DMA] * 2,
)
def _face_k(px, py, pz, f0, f1, f2, ocx, ocy, ocz,
            spx, spy, spz, i0, i1, i2, *rest):
    bufs = [rest[9 * b:9 * b + 9] for b in range(2)]
    obx, oby, obz = rest[18:21]
    sems = rest[21:23]
    w = _wid()
    fbase = w * FT

    @pl.when(lax.axis_index("s") == 0)
    def _stage():
        pltpu.sync_copy(px, spx)
        pltpu.sync_copy(py, spy)
        pltpu.sync_copy(pz, spz)

    pltpu.sync_copy(f0.at[pl.ds(fbase, FT)], i0)
    pltpu.sync_copy(f1.at[pl.ds(fbase, FT)], i1)
    pltpu.sync_copy(f2.at[pl.ds(fbase, FT)], i2)
    plsc.subcore_barrier()

    def srcs(j):
        jsl = pl.ds(j * CHUNK, CHUNK)
        idx0, idx1, idx2 = i0.at[jsl], i1.at[jsl], i2.at[jsl]
        return [spx.at[idx0], spy.at[idx0], spz.at[idx0],
                spx.at[idx1], spy.at[idx1], spz.at[idx1],
                spx.at[idx2], spy.at[idx2], spz.at[idx2]]

    def issue(j, b):
        for s, d in zip(srcs(j), bufs[b]):
            pltpu.async_copy(s, d, sems[b])

    def drain(j, b):
        for s, d in zip(srcs(j), bufs[b]):
            pltpu.make_async_copy(s, d, sems[b]).wait()

    def compute(j, b):
        b0x, b0y, b0z, b1x, b1y, b1z, b2x, b2y, b2z = bufs[b]
        for sb in range(8):
            sl = pl.ds(sb * LANES, LANES)
            ax, ay, az = b0x[sl], b0y[sl], b0z[sl]
            ux, uy, uz = b1x[sl] - ax, b1y[sl] - ay, b1z[sl] - az
            wx, wy, wz = b2x[sl] - ax, b2y[sl] - ay, b2z[sl] - az
            osl = pl.ds(j * CHUNK + sb * LANES, LANES)
            obx[osl] = uy * wz - uz * wy
            oby[osl] = uz * wx - ux * wz
            obz[osl] = ux * wy - uy * wx

    issue(0, 0)

    def body(t, carry):
        j0 = 2 * t
        issue(j0 + 1, 1)
        drain(j0, 0)
        compute(j0, 0)
        pl.when(j0 + 2 < FC)(lambda: issue(j0 + 2, 0))
        drain(j0 + 1, 1)
        compute(j0 + 1, 1)
        return carry

    lax.fori_loop(0, FC // 2, body, 0, unroll=False)
    drain(FC - 1, 0)
    compute(FC - 1, 0)

    dst = pl.ds(fbase, FT)
    pltpu.sync_copy(obx, ocx.at[dst])
    pltpu.sync_copy(oby, ocy.at[dst])
    pltpu.sync_copy(obz, ocz.at[dst])


def _edge_compute(j, bufs, acc):
    (g0x, g0y, g0z, g0a, g1x, g1y, g1z, g1a,
     h0x, h0y, h0z, h1x, h1y, h1z) = bufs
    for sb in range(8):
        sl = pl.ds(sb * LANES, LANES)
        c0x, c0y, c0z, a0 = g0x[sl], g0y[sl], g0z[sl], g0a[sl]
        c1x, c1y, c1z, a1 = g1x[sl], g1y[sl], g1z[sl], g1a[sl]
        ex = h1x[sl] - h0x[sl]
        ey = h1y[sl] - h0y[sl]
        ez = h1z[sl] - h0z[sl]
        ll = ex * ex + ey * ey + ez * ez
        dot = c0x * c1x + c0y * c1y + c0z * c1z
        gx = c0y * c1z - c0z * c1y
        gy = c0z * c1x - c0x * c1z
        gz = c0x * c1y - c0y * c1x
        sn = ex * gx + ey * gy + ez * gz
        s = sn * _rsqrt(ll)
        th2 = _atan2_sq(s, dot)
        acc = acc + th2 * ll / (4.0 * (a0 + a1))
    return acc


@functools.partial(
    pl.kernel,
    out_type=jax.ShapeDtypeStruct((NW * LANES,), jnp.float32),
    mesh=_MESH,
    scratch_types=[pltpu.VMEM_SHARED((NV,), jnp.float32)] * 3
    + [pltpu.VMEM_SHARED((FP,), jnp.float32)] * 4
    + [pltpu.VMEM((ET,), jnp.int32)] * 4
    + [_F32B] * 28
    + [pltpu.VMEM((LANES,), jnp.float32)]
    + [pltpu.SemaphoreType.DMA] * 2,
)
def _edge_k(px, py, pz, cx, cy, cz, af, jc0, jc1, je0, je1, part_out,
            spx, spy, spz, scx, scy, scz, saf,
            ic0, ic1, ie0, ie1, *rest):
    bufs = [rest[14 * b:14 * b + 14] for b in range(2)]
    accb = rest[28]
    sems = rest[29:31]
    w = _wid()
    ebase = w * ET

    @pl.when(lax.axis_index("s") == 0)
    def _stage():
        pltpu.sync_copy(px, spx)
        pltpu.sync_copy(py, spy)
        pltpu.sync_copy(pz, spz)
        pltpu.sync_copy(cx, scx)
        pltpu.sync_copy(cy, scy)
        pltpu.sync_copy(cz, scz)
        pltpu.sync_copy(af, saf)

    pltpu.sync_copy(jc0.at[pl.ds(ebase, ET)], ic0)
    pltpu.sync_copy(jc1.at[pl.ds(ebase, ET)], ic1)
    pltpu.sync_copy(je0.at[pl.ds(ebase, ET)], ie0)
    pltpu.sync_copy(je1.at[pl.ds(ebase, ET)], ie1)
    plsc.subcore_barrier()

    def srcs(j):
        jsl = pl.ds(j * CHUNK, CHUNK)
        ix0, ix1, iv0, iv1 = ic0.at[jsl], ic1.at[jsl], ie0.at[jsl], ie1.at[jsl]
        return [scx.at[ix0], scy.at[ix0], scz.at[ix0], saf.at[ix0],
                scx.at[ix1], scy.at[ix1], scz.at[ix1], saf.at[ix1],
                spx.at[iv0], spy.at[iv0], spz.at[iv0],
                spx.at[iv1], spy.at[iv1], spz.at[iv1]]

    def issue(j, b):
        for s, d in zip(srcs(j), bufs[b]):
            pltpu.async_copy(s, d, sems[b])

    def drain(j, b):
        for s, d in zip(srcs(j), bufs[b]):
            pltpu.make_async_copy(s, d, sems[b]).wait()

    issue(0, 0)

    def body(t, acc):
        j0 = 2 * t
        issue(j0 + 1, 1)
        drain(j0, 0)
        acc = _edge_compute(j0, bufs[0], acc)
        pl.when(j0 + 2 < EC)(lambda: issue(j0 + 2, 0))
        drain(j0 + 1, 1)
        acc = _edge_compute(j0 + 1, bufs[1], acc)
        return acc

    acc = lax.fori_loop(0, EC // 2, body, jnp.zeros((LANES,), jnp.float32),
                        unroll=False)
    drain(EC - 1, 0)
    acc = _edge_compute(EC - 1, bufs[0], acc)
    accb[...] = acc
    pltpu.sync_copy(accb, part_out.at[pl.ds(w * LANES, LANES)])


def kernel(pred_pos, faces, f_connectivity, f_connectivity_edges, f_area,
           bending_coeff):
    px, py, pz = pred_pos[:, 0], pred_pos[:, 1], pred_pos[:, 2]
    ft = jnp.pad(faces.T.astype(jnp.int32), ((0, 0), (0, FP - NF)))
    af = jnp.pad(f_area[:, 0], (0, FP - NF))
    fct = jnp.pad(f_connectivity.T.astype(jnp.int32), ((0, 0), (0, EP - NE)))
    fet = jnp.pad(f_connectivity_edges.T.astype(jnp.int32),
                  ((0, 0), (0, EP - NE)))

    cx, cy, cz = _face_k(px, py, pz, ft[0], ft[1], ft[2])
    parts = _edge_k(px, py, pz, cx, cy, cz, af,
                    fct[0], fct[1], fet[0], fet[1])
    return jnp.sum(parts) * (bending_coeff[0] * 0.5)


# speedup vs baseline: 2.0772x; 1.0044x over previous
import functools

import jax
import jax.numpy as jnp
from jax import lax
from jax.experimental import pallas as pl
from jax.experimental.pallas import tpu as pltpu, tpu_sc as plsc

NV = 50000
NF = 100000
NE = 150000

NC = 2
NS = 16
NW = NC * NS
LANES = 16
CHUNK = 128

FC = 25
FT = FC * CHUNK
FP = NW * FT
EC = 37
ET = EC * CHUNK
EP = NW * ET

_ATAN = (0.999998017, -0.333060167, 0.196054925, -0.122270662,
         0.0585597433, -0.0138876227)
PI = 3.14159265358979
PIO2 = PI / 2

_MESH = plsc.VectorSubcoreMesh(core_axis_name="c", subcore_axis_name="s")


def _rsqrt(x):
    i = lax.bitcast_convert_type(x, jnp.int32)
    i = jnp.int32(0x5F3759DF) - (i >> 1)
    y = lax.bitcast_convert_type(i, jnp.float32)
    for _ in range(3):
        y = y * (1.5 - 0.5 * x * y * y)
    return y


def _atan2_sq(s, d):
    p, q = jnp.abs(d), jnp.abs(s)
    mx, mn = jnp.maximum(p, q), jnp.minimum(p, q)
    z = mn / jnp.maximum(mx, 1e-30)
    z2 = z * z
    t = jnp.float32(_ATAN[5])
    for c in _ATAN[4::-1]:
        t = t * z2 + c
    t = t * z
    t = jnp.where(q > p, PIO2 - t, t)
    t = jnp.where(d < 0.0, PI - t, t)
    return t * t


def _wid():
    return lax.axis_index("s") * NC + lax.axis_index("c")


_F32B = pltpu.VMEM((CHUNK,), jnp.float32)


@functools.partial(
    pl.kernel,
    out_type=(jax.ShapeDtypeStruct((FP,), jnp.float32),) * 3,
    mesh=_MESH,
    scratch_types=[pltpu.VMEM_SHARED((NV,), jnp.float32)] * 3
    + [pltpu.VMEM((FT,), jnp.int32)] * 3
    + [_F32B] * 18
    + [_F32B] * 6
    + [pltpu.SemaphoreType.DMA] * 2,
)
def _face_k(px, py, pz, f0, f1, f2, ocx, ocy, ocz,
            spx, spy, spz, i0, i1, i2, *rest):
    bufs = [rest[9 * b:9 * b + 9] for b in range(2)]
    obufs = [rest[18 + 3 * b:21 + 3 * b] for b in range(2)]
    sems = rest[24:26]
    w = _wid()
    fbase = w * FT

    @pl.when(lax.axis_index("s") == 0)
    def _stage():
        pltpu.sync_copy(px, spx)
        pltpu.sync_copy(py, spy)
        pltpu.sync_copy(pz, spz)

    pltpu.sync_copy(f0.at[pl.ds(fbase, FT)], i0)
    pltpu.sync_copy(f1.at[pl.ds(fbase, FT)], i1)
    pltpu.sync_copy(f2.at[pl.ds(fbase, FT)], i2)
    plsc.subcore_barrier()

    def srcs(j):
        jsl = pl.ds(j * CHUNK, CHUNK)
        idx0, idx1, idx2 = i0.at[jsl], i1.at[jsl], i2.at[jsl]
        return [spx.at[idx0], spy.at[idx0], spz.at[idx0],
                spx.at[idx1], spy.at[idx1], spz.at[idx1],
                spx.at[idx2], spy.at[idx2], spz.at[idx2]]

    def issue(j, b):
        for s, d in zip(srcs(j), bufs[b]):
            pltpu.async_copy(s, d, sems[b])

    def drain(j, b):
        for s, d in zip(srcs(j), bufs[b]):
            pltpu.make_async_copy(s, d, sems[b]).wait()

    def compute(j, b):
        b0x, b0y, b0z, b1x, b1y, b1z, b2x, b2y, b2z = bufs[b]
        sx, sy, sz = obufs[b]
        for sb in range(8):
            sl = pl.ds(sb * LANES, LANES)
            ax, ay, az = b0x[sl], b0y[sl], b0z[sl]
            ux, uy, uz = b1x[sl] - ax, b1y[sl] - ay, b1z[sl] - az
            wx, wy, wz = b2x[sl] - ax, b2y[sl] - ay, b2z[sl] - az
            sx[sl] = uy * wz - uz * wy
            sy[sl] = uz * wx - ux * wz
            sz[sl] = ux * wy - uy * wx
        dst = pl.ds(fbase + j * CHUNK, CHUNK)
        pltpu.sync_copy(sx, ocx.at[dst])
        pltpu.sync_copy(sy, ocy.at[dst])
        pltpu.sync_copy(sz, ocz.at[dst])

    issue(0, 0)

    def body(t, carry):
        j0 = 2 * t
        issue(j0 + 1, 1)
        drain(j0, 0)
        compute(j0, 0)
        pl.when(j0 + 2 < FC)(lambda: issue(j0 + 2, 0))
        drain(j0 + 1, 1)
        compute(j0 + 1, 1)
        return carry

    lax.fori_loop(0, FC // 2, body, 0, unroll=False)
    drain(FC - 1, 0)
    compute(FC - 1, 0)


def _edge_compute(j, bufs, acc):
    (g0x, g0y, g0z, g0a, g1x, g1y, g1z, g1a,
     h0x, h0y, h0z, h1x, h1y, h1z) = bufs
    for sb in range(8):
        sl = pl.ds(sb * LANES, LANES)
        c0x, c0y, c0z, a0 = g0x[sl], g0y[sl], g0z[sl], g0a[sl]
        c1x, c1y, c1z, a1 = g1x[sl], g1y[sl], g1z[sl], g1a[sl]
        ex = h1x[sl] - h0x[sl]
        ey = h1y[sl] - h0y[sl]
        ez = h1z[sl] - h0z[sl]
        ll = ex * ex + ey * ey + ez * ez
        dot = c0x * c1x + c0y * c1y + c0z * c1z
        gx = c0y * c1z - c0z * c1y
        gy = c0z * c1x - c0x * c1z
        gz = c0x * c1y - c0y * c1x
        sn = ex * gx + ey * gy + ez * gz
        s = sn * _rsqrt(ll)
        th2 = _atan2_sq(s, dot)
        acc = acc + th2 * ll / (4.0 * (a0 + a1))
    return acc


@functools.partial(
    pl.kernel,
    out_type=jax.ShapeDtypeStruct((NW * LANES,), jnp.float32),
    mesh=_MESH,
    scratch_types=[pltpu.VMEM_SHARED((NV,), jnp.float32)] * 3
    + [pltpu.VMEM_SHARED((FP,), jnp.float32)] * 4
    + [pltpu.VMEM((ET,), jnp.int32)] * 4
    + [_F32B] * 28
    + [pltpu.VMEM((LANES,), jnp.float32)]
    + [pltpu.SemaphoreType.DMA] * 2,
)
def _edge_k(px, py, pz, cx, cy, cz, af, jc0, jc1, je0, je1, part_out,
            spx, spy, spz, scx, scy, scz, saf,
            ic0, ic1, ie0, ie1, *rest):
    bufs = [rest[14 * b:14 * b + 14] for b in range(2)]
    accb = rest[28]
    sems = rest[29:31]
    w = _wid()
    ebase = w * ET

    @pl.when(lax.axis_index("s") == 0)
    def _stage():
        pltpu.sync_copy(px, spx)
        pltpu.sync_copy(py, spy)
        pltpu.sync_copy(pz, spz)
        pltpu.sync_copy(cx, scx)
        pltpu.sync_copy(cy, scy)
        pltpu.sync_copy(cz, scz)
        pltpu.sync_copy(af, saf)

    pltpu.sync_copy(jc0.at[pl.ds(ebase, ET)], ic0)
    pltpu.sync_copy(jc1.at[pl.ds(ebase, ET)], ic1)
    pltpu.sync_copy(je0.at[pl.ds(ebase, ET)], ie0)
    pltpu.sync_copy(je1.at[pl.ds(ebase, ET)], ie1)
    plsc.subcore_barrier()

    def srcs(j):
        jsl = pl.ds(j * CHUNK, CHUNK)
        ix0, ix1, iv0, iv1 = ic0.at[jsl], ic1.at[jsl], ie0.at[jsl], ie1.at[jsl]
        return [scx.at[ix0], scy.at[ix0], scz.at[ix0], saf.at[ix0],
                scx.at[ix1], scy.at[ix1], scz.at[ix1], saf.at[ix1],
                spx.at[iv0], spy.at[iv0], spz.at[iv0],
                spx.at[iv1], spy.at[iv1], spz.at[iv1]]

    def issue(j, b):
        for s, d in zip(srcs(j), bufs[b]):
            pltpu.async_copy(s, d, sems[b])

    def drain(j, b):
        for s, d in zip(srcs(j), bufs[b]):
            pltpu.make_async_copy(s, d, sems[b]).wait()

    issue(0, 0)

    def body(t, acc):
        j0 = 2 * t
        issue(j0 + 1, 1)
        drain(j0, 0)
        acc = _edge_compute(j0, bufs[0], acc)
        pl.when(j0 + 2 < EC)(lambda: issue(j0 + 2, 0))
        drain(j0 + 1, 1)
        acc = _edge_compute(j0 + 1, bufs[1], acc)
        return acc

    acc = lax.fori_loop(0, EC // 2, body, jnp.zeros((LANES,), jnp.float32),
                        unroll=False)
    drain(EC - 1, 0)
    acc = _edge_compute(EC - 1, bufs[0], acc)
    accb[...] = acc
    pltpu.sync_copy(accb, part_out.at[pl.ds(w * LANES, LANES)])


def kernel(pred_pos, faces, f_connectivity, f_connectivity_edges, f_area,
           bending_coeff):
    px, py, pz = pred_pos[:, 0], pred_pos[:, 1], pred_pos[:, 2]
    ft = jnp.pad(faces.T.astype(jnp.int32), ((0, 0), (0, FP - NF)))
    af = jnp.pad(f_area[:, 0], (0, FP - NF))
    fct = jnp.pad(f_connectivity.T.astype(jnp.int32), ((0, 0), (0, EP - NE)))
    fet = jnp.pad(f_connectivity_edges.T.astype(jnp.int32),
                  ((0, 0), (0, EP - NE)))

    cx, cy, cz = _face_k(px, py, pz, ft[0], ft[1], ft[2])
    parts = _edge_k(px, py, pz, cx, cy, cz, af,
                    fct[0], fct[1], fet[0], fet[1])
    return jnp.sum(parts) * (bending_coeff[0] * 0.5)
